# Initial kernel scaffold; baseline (speedup 1.0000x reference)
import jax
import jax.numpy as jnp
from jax.experimental import pallas as pl


def kernel(values, edge_index, W, bias):
    raise NotImplementedError("write your pallas kernel here")


# speedup vs baseline: 2.6767x; 2.6767x over previous
import functools

import jax
import jax.numpy as jnp
from jax import lax
from jax.experimental import pallas as pl
from jax.experimental.pallas import tpu as pltpu
from jax.experimental.pallas import tpu_sc as plsc

N_ROWS = 10000
E = 320000
D = 128
CH = 128
NB = E // CH
NS = 16
NC = 2

CPT = NB // NS
CREM = NB - CPT * NS
ROWS_PT = N_ROWS // NS
RQ = 5
RS = ROWS_PT // RQ

NW = NC * NS
GPT = NB // NW
GREM = NB - GPT * NW


def _mesh():
    return plsc.VectorSubcoreMesh(core_axis_name="c", subcore_axis_name="s")


def _pools_body(values, idx2d, out, idxbuf, vbuf0, vbuf1, pool_sh,
                sem0, sem1):
    c = lax.axis_index("c")
    s = lax.axis_index("s")

    def zero_row(r, carry):
        for l in range(D // 16):
            vbuf0[r, pl.ds(l * 16, 16)] = jnp.zeros((16,), jnp.float32)
        return carry
    lax.fori_loop(0, CH, zero_row, 0)
    for q in range(RQ):
        pltpu.sync_copy(vbuf0.at[pl.ds(0, RS)],
                        pool_sh.at[pl.ds(s * ROWS_PT + q * RS, RS)])
    plsc.subcore_barrier()

    base = s * CPT

    def mk_load(j, vb, islot, sem):
        di = pltpu.make_async_copy(idx2d.at[c, j], idxbuf.at[islot], sem)
        dv = pltpu.make_async_copy(values.at[pl.ds(j * CH, CH)], vb, sem)
        return di, dv

    def start(j, vb, islot, sem):
        di, dv = mk_load(j, vb, islot, sem)
        di.start()
        dv.start()

    def wait(j, vb, islot, sem):
        di, dv = mk_load(j, vb, islot, sem)
        di.wait()
        dv.wait()

    start(base + 0, vbuf0, 0, sem0)
    start(base + 1, vbuf1, 1, sem1)

    def step(u, carry):
        j0 = base + 2 * u
        wait(j0, vbuf0, 0, sem0)
        pltpu.sync_copy(vbuf0, pool_sh.at[idxbuf.at[0]], add=True)

        @pl.when(u < CPT // 2 - 1)
        def _():
            start(j0 + 2, vbuf0, 0, sem0)

        wait(j0 + 1, vbuf1, 1, sem1)
        pltpu.sync_copy(vbuf1, pool_sh.at[idxbuf.at[1]], add=True)

        @pl.when(u < CPT // 2 - 1)
        def _():
            start(j0 + 3, vbuf1, 1, sem1)
        return carry
    lax.fori_loop(0, CPT // 2, step, 0)

    @pl.when(s < CREM)
    def _():
        jx = NS * CPT + s
        start(jx, vbuf0, 0, sem0)
        wait(jx, vbuf0, 0, sem0)
        pltpu.sync_copy(vbuf0, pool_sh.at[idxbuf.at[0]], add=True)

    plsc.subcore_barrier()

    for q in range(RQ):
        r0 = s * ROWS_PT + q * RS
        pltpu.sync_copy(pool_sh.at[pl.ds(r0, RS)], vbuf0.at[pl.ds(0, RS)])
        pltpu.sync_copy(vbuf0.at[pl.ds(0, RS)], out.at[c, pl.ds(r0, RS)])


_pools_call = functools.partial(
    pl.kernel,
    out_type=jax.ShapeDtypeStruct((NC, N_ROWS, D), jnp.float32),
    mesh=_mesh(),
    compiler_params=pltpu.CompilerParams(use_tc_tiling_on_sc=False),
    scratch_types=[
        pltpu.VMEM((2, CH), jnp.int32),
        pltpu.VMEM((CH, D), jnp.float32),
        pltpu.VMEM((CH, D), jnp.float32),
        pltpu.VMEM_SHARED((N_ROWS, D), jnp.float32),
        pltpu.SemaphoreType.DMA,
        pltpu.SemaphoreType.DMA,
    ],
)(_pools_body)


_SB = 10
_SR = N_ROWS // _SB


def _small_body(rp_ref, cp_ref, w1_ref, w2_ref, w3_ref, bias_ref,
                p1_ref, p2_ref, tvec_ref):
    i = pl.program_id(0)
    rp = rp_ref[...]
    p1_ref[...] = jax.lax.dot_general(
        rp, w1_ref[...], (((1,), (0,)), ((), ())),
        precision=lax.Precision.HIGHEST, preferred_element_type=jnp.float32)
    p2_ref[...] = jax.lax.dot_general(
        cp_ref[...], w2_ref[...], (((1,), (0,)), ((), ())),
        precision=lax.Precision.HIGHEST, preferred_element_type=jnp.float32)

    @pl.when(i == 0)
    def _():
        tvec_ref[...] = jnp.zeros_like(tvec_ref)

    tvec_ref[...] += jnp.sum(rp, axis=0, keepdims=True)

    @pl.when(i == _SB - 1)
    def _():
        tot = tvec_ref[...]
        bsum = bias_ref[0] + bias_ref[1] + bias_ref[2] + bias_ref[3]
        tvec_ref[...] = jax.lax.dot_general(
            tot, w3_ref[...], (((1,), (0,)), ((), ())),
            precision=lax.Precision.HIGHEST,
            preferred_element_type=jnp.float32) + bsum


def _small_call(rp, cp, w1, w2, w3, bias):
    return pl.pallas_call(
        _small_body,
        grid=(_SB,),
        in_specs=[
            pl.BlockSpec((_SR, D), lambda i: (i, 0)),
            pl.BlockSpec((_SR, D), lambda i: (i, 0)),
            pl.BlockSpec((D, D), lambda i: (0, 0)),
            pl.BlockSpec((D, D), lambda i: (0, 0)),
            pl.BlockSpec((D, D), lambda i: (0, 0)),
            pl.BlockSpec(memory_space=pltpu.SMEM),
        ],
        out_specs=[
            pl.BlockSpec((_SR, D), lambda i: (i, 0)),
            pl.BlockSpec((_SR, D), lambda i: (i, 0)),
            pl.BlockSpec((1, D), lambda i: (0, 0)),
        ],
        out_shape=[
            jax.ShapeDtypeStruct((N_ROWS, D), jnp.float32),
            jax.ShapeDtypeStruct((N_ROWS, D), jnp.float32),
            jax.ShapeDtypeStruct((1, D), jnp.float32),
        ],
    )(rp, cp, w1, w2, w3, bias)


def _gather_body(idx2d, p1, p2, g_out, idxr, idxc, buf1, buf2, semA, semB):
    c = lax.axis_index("c")
    s = lax.axis_index("s")
    wid = s * NC + c
    base = wid * GPT

    pltpu.sync_copy(idx2d.at[0, pl.ds(base, GPT)], idxr.at[pl.ds(0, GPT)])
    pltpu.sync_copy(idx2d.at[1, pl.ds(base, GPT)], idxc.at[pl.ds(0, GPT)])

    @pl.when(wid < GREM)
    def _():
        pltpu.sync_copy(idx2d.at[0, pl.ds(NW * GPT + wid, 1)],
                        idxr.at[pl.ds(GPT, 1)])
        pltpu.sync_copy(idx2d.at[1, pl.ds(NW * GPT + wid, 1)],
                        idxc.at[pl.ds(GPT, 1)])

    def do_chunk(t, j):
        d1 = pltpu.make_async_copy(p1.at[idxr.at[t]], buf1, semA)
        d2 = pltpu.make_async_copy(p2.at[idxc.at[t]], buf2, semB)
        d1.start()
        d2.start()
        d1.wait()
        d2.wait()

        def add_row(r, carry):
            for l in range(D // 16):
                sl = pl.ds(l * 16, 16)
                buf1[r, sl] = buf1[r, sl] + buf2[r, sl]
            return carry
        lax.fori_loop(0, CH, add_row, 0)
        pltpu.sync_copy(buf1, g_out.at[pl.ds(j * CH, CH)])

    def step(t, carry):
        do_chunk(t, base + t)
        return carry
    lax.fori_loop(0, GPT, step, 0)

    @pl.when(wid < GREM)
    def _():
        do_chunk(GPT, NW * GPT + wid)


_gather_call = functools.partial(
    pl.kernel,
    out_type=jax.ShapeDtypeStruct((E, D), jnp.float32),
    mesh=_mesh(),
    compiler_params=pltpu.CompilerParams(use_tc_tiling_on_sc=False),
    scratch_types=[
        pltpu.VMEM((GPT + 1, CH), jnp.int32),
        pltpu.VMEM((GPT + 1, CH), jnp.int32),
        pltpu.VMEM((CH, D), jnp.float32),
        pltpu.VMEM((CH, D), jnp.float32),
        pltpu.SemaphoreType.DMA,
        pltpu.SemaphoreType.DMA,
    ],
)(_gather_body)


_MB = 512
_MG = E // _MB


def _main_body(v_ref, g_ref, tvec_ref, w0_ref, y_ref):
    y_ref[...] = jax.lax.dot_general(
        v_ref[...], w0_ref[...], (((1,), (0,)), ((), ())),
        precision=lax.Precision.HIGHEST,
        preferred_element_type=jnp.float32) + g_ref[...] + tvec_ref[...]


def _main_call(values, g, tvec, w0):
    return pl.pallas_call(
        _main_body,
        grid=(_MG,),
        in_specs=[
            pl.BlockSpec((_MB, D), lambda i: (i, 0)),
            pl.BlockSpec((_MB, D), lambda i: (i, 0)),
            pl.BlockSpec((1, D), lambda i: (0, 0)),
            pl.BlockSpec((D, D), lambda i: (0, 0)),
        ],
        out_specs=pl.BlockSpec((_MB, D), lambda i: (i, 0)),
        out_shape=jax.ShapeDtypeStruct((E, D), jnp.float32),
    )(values, g, tvec, w0)


def kernel(values, edge_index, W, bias):
    idx2d = edge_index.reshape(2, NB, CH)
    pools = _pools_call(values, idx2d)
    p1, p2, tvec = _small_call(pools[0], pools[1], W[1], W[2], W[3], bias)
    g = _gather_call(idx2d, p1, p2)
    return _main_call(values, g, tvec, W[0])


# speedup vs baseline: 3.0300x; 1.1320x over previous
import functools

import jax
import jax.numpy as jnp
from jax import lax
from jax.experimental import pallas as pl
from jax.experimental.pallas import tpu as pltpu
from jax.experimental.pallas import tpu_sc as plsc

N_ROWS = 10000
E = 320000
D = 128
CH = 128
NB = E // CH
NS = 16
NC = 2

CPT = NB // NS
CREM = NB - CPT * NS
ROWS_PT = N_ROWS // NS
RQ = 5
RS = ROWS_PT // RQ

NW = NC * NS
GPT = NB // NW
GREM = NB - GPT * NW


def _mesh():
    return plsc.VectorSubcoreMesh(core_axis_name="c", subcore_axis_name="s")


def _pools_body(values, idx2d, out, idxbuf, vbuf0, vbuf1, pool_sh,
                sem0, sem1):
    c = lax.axis_index("c")
    s = lax.axis_index("s")

    def zero_row(r, carry):
        for l in range(D // 16):
            vbuf0[r, pl.ds(l * 16, 16)] = jnp.zeros((16,), jnp.float32)
        return carry
    lax.fori_loop(0, CH, zero_row, 0)
    for q in range(RQ):
        pltpu.sync_copy(vbuf0.at[pl.ds(0, RS)],
                        pool_sh.at[pl.ds(s * ROWS_PT + q * RS, RS)])
    plsc.subcore_barrier()

    base = s * CPT

    def mk_load(j, vb, islot, sem):
        di = pltpu.make_async_copy(idx2d.at[c, j], idxbuf.at[islot], sem)
        dv = pltpu.make_async_copy(values.at[pl.ds(j * CH, CH)], vb, sem)
        return di, dv

    def start(j, vb, islot, sem):
        di, dv = mk_load(j, vb, islot, sem)
        di.start()
        dv.start()

    def wait(j, vb, islot, sem):
        di, dv = mk_load(j, vb, islot, sem)
        di.wait()
        dv.wait()

    start(base + 0, vbuf0, 0, sem0)
    start(base + 1, vbuf1, 1, sem1)

    def step(u, carry):
        j0 = base + 2 * u
        wait(j0, vbuf0, 0, sem0)
        pltpu.sync_copy(vbuf0, pool_sh.at[idxbuf.at[0]], add=True)

        @pl.when(u < CPT // 2 - 1)
        def _():
            start(j0 + 2, vbuf0, 0, sem0)

        wait(j0 + 1, vbuf1, 1, sem1)
        pltpu.sync_copy(vbuf1, pool_sh.at[idxbuf.at[1]], add=True)

        @pl.when(u < CPT // 2 - 1)
        def _():
            start(j0 + 3, vbuf1, 1, sem1)
        return carry
    lax.fori_loop(0, CPT // 2, step, 0)

    @pl.when(s < CREM)
    def _():
        jx = NS * CPT + s
        start(jx, vbuf0, 0, sem0)
        wait(jx, vbuf0, 0, sem0)
        pltpu.sync_copy(vbuf0, pool_sh.at[idxbuf.at[0]], add=True)

    plsc.subcore_barrier()

    for q in range(RQ):
        r0 = s * ROWS_PT + q * RS
        pltpu.sync_copy(pool_sh.at[pl.ds(r0, RS)], vbuf0.at[pl.ds(0, RS)])
        pltpu.sync_copy(vbuf0.at[pl.ds(0, RS)], out.at[c, pl.ds(r0, RS)])


_pools_call = functools.partial(
    pl.kernel,
    out_type=jax.ShapeDtypeStruct((NC, N_ROWS, D), jnp.float32),
    mesh=_mesh(),
    compiler_params=pltpu.CompilerParams(use_tc_tiling_on_sc=False),
    scratch_types=[
        pltpu.VMEM((2, CH), jnp.int32),
        pltpu.VMEM((CH, D), jnp.float32),
        pltpu.VMEM((CH, D), jnp.float32),
        pltpu.VMEM_SHARED((N_ROWS, D), jnp.float32),
        pltpu.SemaphoreType.DMA,
        pltpu.SemaphoreType.DMA,
    ],
)(_pools_body)


_SB = 10
_SR = N_ROWS // _SB


def _small_body(rp_ref, cp_ref, w1_ref, w2_ref, w3_ref, bias_ref,
                p1_ref, p2_ref, tvec_ref):
    i = pl.program_id(0)
    rp = rp_ref[...]
    p1_ref[...] = jax.lax.dot_general(
        rp, w1_ref[...], (((1,), (0,)), ((), ())),
        precision=lax.Precision.HIGHEST, preferred_element_type=jnp.float32)
    p2_ref[...] = jax.lax.dot_general(
        cp_ref[...], w2_ref[...], (((1,), (0,)), ((), ())),
        precision=lax.Precision.HIGHEST, preferred_element_type=jnp.float32)

    @pl.when(i == 0)
    def _():
        tvec_ref[...] = jnp.zeros_like(tvec_ref)

    tvec_ref[...] += jnp.sum(rp, axis=0, keepdims=True)

    @pl.when(i == _SB - 1)
    def _():
        tot = tvec_ref[...]
        bsum = bias_ref[0] + bias_ref[1] + bias_ref[2] + bias_ref[3]
        tvec_ref[...] = jax.lax.dot_general(
            tot, w3_ref[...], (((1,), (0,)), ((), ())),
            precision=lax.Precision.HIGHEST,
            preferred_element_type=jnp.float32) + bsum


def _small_call(rp, cp, w1, w2, w3, bias):
    return pl.pallas_call(
        _small_body,
        grid=(_SB,),
        in_specs=[
            pl.BlockSpec((_SR, D), lambda i: (i, 0)),
            pl.BlockSpec((_SR, D), lambda i: (i, 0)),
            pl.BlockSpec((D, D), lambda i: (0, 0)),
            pl.BlockSpec((D, D), lambda i: (0, 0)),
            pl.BlockSpec((D, D), lambda i: (0, 0)),
            pl.BlockSpec(memory_space=pltpu.SMEM),
        ],
        out_specs=[
            pl.BlockSpec((_SR, D), lambda i: (i, 0)),
            pl.BlockSpec((_SR, D), lambda i: (i, 0)),
            pl.BlockSpec((1, D), lambda i: (0, 0)),
        ],
        out_shape=[
            jax.ShapeDtypeStruct((N_ROWS, D), jnp.float32),
            jax.ShapeDtypeStruct((N_ROWS, D), jnp.float32),
            jax.ShapeDtypeStruct((1, D), jnp.float32),
        ],
    )(rp, cp, w1, w2, w3, bias)


def _gather_body(idx2d, p1, p2, g_out, idxr, idxc, buf1, buf2, sbuf,
                 semA0, semA1, semB0, semB1, semS0, semS1):
    c = lax.axis_index("c")
    s = lax.axis_index("s")
    wid = s * NC + c
    base = wid * GPT
    semA = (semA0, semA1)
    semB = (semB0, semB1)
    semS = (semS0, semS1)

    pltpu.sync_copy(idx2d.at[0, pl.ds(base, GPT)], idxr.at[pl.ds(0, GPT)])
    pltpu.sync_copy(idx2d.at[1, pl.ds(base, GPT)], idxc.at[pl.ds(0, GPT)])

    @pl.when(wid < GREM)
    def _():
        pltpu.sync_copy(idx2d.at[0, pl.ds(NW * GPT + wid, 1)],
                        idxr.at[pl.ds(GPT, 1)])
        pltpu.sync_copy(idx2d.at[1, pl.ds(NW * GPT + wid, 1)],
                        idxc.at[pl.ds(GPT, 1)])

    def g_descs(t, k):
        return (pltpu.make_async_copy(p1.at[idxr.at[t]], buf1.at[k], semA[k]),
                pltpu.make_async_copy(p2.at[idxc.at[t]], buf2.at[k], semB[k]))

    def g_start(t, k):
        d1, d2 = g_descs(t, k)
        d1.start()
        d2.start()

    def g_wait(t, k):
        d1, d2 = g_descs(t, k)
        d1.wait()
        d2.wait()

    def s_desc(j, k):
        return pltpu.make_async_copy(sbuf.at[k], g_out.at[pl.ds(j * CH, CH)],
                                     semS[k])

    def add_chunk(k):
        def add_row(r, carry):
            for l in range(D // 16):
                sl = pl.ds(l * 16, 16)
                sbuf[k, r, sl] = buf1[k, r, sl] + buf2[k, r, sl]
            return carry
        lax.fori_loop(0, CH, add_row, 0)

    g_start(0, 0)
    g_start(1, 1)

    def step(u, carry):
        for k in range(2):
            t = 2 * u + k
            g_wait(t, k)

            @pl.when(u > 0)
            def _():
                s_desc(base + t - 2, k).wait()
            add_chunk(k)

            @pl.when(u < GPT // 2 - 1)
            def _():
                g_start(t + 2, k)
            s_desc(base + t, k).start()
        return carry
    lax.fori_loop(0, GPT // 2, step, 0)
    s_desc(base + GPT - 2, 0).wait()
    s_desc(base + GPT - 1, 1).wait()

    @pl.when(wid < GREM)
    def _():
        jx = NW * GPT + wid
        g_start(GPT, 0)
        g_wait(GPT, 0)
        add_chunk(0)
        s_desc(jx, 0).start()
        s_desc(jx, 0).wait()


_gather_call = functools.partial(
    pl.kernel,
    out_type=jax.ShapeDtypeStruct((E, D), jnp.float32),
    mesh=_mesh(),
    compiler_params=pltpu.CompilerParams(use_tc_tiling_on_sc=False),
    scratch_types=[
        pltpu.VMEM((GPT + 1, CH), jnp.int32),
        pltpu.VMEM((GPT + 1, CH), jnp.int32),
        pltpu.VMEM((2, CH, D), jnp.float32),
        pltpu.VMEM((2, CH, D), jnp.float32),
        pltpu.VMEM((2, CH, D), jnp.float32),
        pltpu.SemaphoreType.DMA,
        pltpu.SemaphoreType.DMA,
        pltpu.SemaphoreType.DMA,
        pltpu.SemaphoreType.DMA,
        pltpu.SemaphoreType.DMA,
        pltpu.SemaphoreType.DMA,
    ],
)(_gather_body)


_MB = 512
_MG = E // _MB


def _main_body(v_ref, g_ref, tvec_ref, w0_ref, y_ref):
    y_ref[...] = jax.lax.dot_general(
        v_ref[...], w0_ref[...], (((1,), (0,)), ((), ())),
        precision=lax.Precision.HIGHEST,
        preferred_element_type=jnp.float32) + g_ref[...] + tvec_ref[...]


def _main_call(values, g, tvec, w0):
    return pl.pallas_call(
        _main_body,
        grid=(_MG,),
        in_specs=[
            pl.BlockSpec((_MB, D), lambda i: (i, 0)),
            pl.BlockSpec((_MB, D), lambda i: (i, 0)),
            pl.BlockSpec((1, D), lambda i: (0, 0)),
            pl.BlockSpec((D, D), lambda i: (0, 0)),
        ],
        out_specs=pl.BlockSpec((_MB, D), lambda i: (i, 0)),
        out_shape=jax.ShapeDtypeStruct((E, D), jnp.float32),
    )(values, g, tvec, w0)


def kernel(values, edge_index, W, bias):
    idx2d = edge_index.reshape(2, NB, CH)
    pools = _pools_call(values, idx2d)
    p1, p2, tvec = _small_call(pools[0], pools[1], W[1], W[2], W[3], bias)
    g = _gather_call(idx2d, p1, p2)
    return _main_call(values, g, tvec, W[0])


# speedup vs baseline: 4.0000x; 1.3201x over previous
import functools

import jax
import jax.numpy as jnp
from jax import lax
from jax.experimental import pallas as pl
from jax.experimental.pallas import tpu as pltpu
from jax.experimental.pallas import tpu_sc as plsc

N_ROWS = 10000
E = 320000
D = 128
CH = 128
NB = E // CH
NS = 16
NC = 2

CPT = NB // NS
CREM = NB - CPT * NS
ROWS_PT = N_ROWS // NS
RQ = 5
RS = ROWS_PT // RQ

NW = NC * NS
GPT = NB // NW
GREM = NB - GPT * NW


def _mesh():
    return plsc.VectorSubcoreMesh(core_axis_name="c", subcore_axis_name="s")


DH = D // 2


def _pools_body(values, idx2d, out, totp, idxbuf, vbuf, acc, pool_sh,
                semL0, semL1, semL2, semC0, semC1, semC2):
    c = lax.axis_index("c")
    s = lax.axis_index("s")
    semL = (semL0, semL1, semL2)
    semC = (semC0, semC1, semC2)
    f0 = c * DH

    def zero_row(r, carry):
        for l in range(DH // 16):
            vbuf[0, r, pl.ds(l * 16, 16)] = jnp.zeros((16,), jnp.float32)
        return carry
    lax.fori_loop(0, CH, zero_row, 0)
    for p in range(2):
        for q in range(RQ):
            pltpu.sync_copy(vbuf.at[0, pl.ds(0, RS)],
                            pool_sh.at[p, pl.ds(s * ROWS_PT + q * RS, RS)])
    plsc.subcore_barrier()

    base = s * CPT

    def load_descs(j, k):
        return (pltpu.make_async_copy(idx2d.at[:, j], idxbuf.at[k], semL[k]),
                pltpu.make_async_copy(
                    values.at[pl.ds(j * CH, CH), pl.ds(f0, DH)],
                    vbuf.at[k], semL[k]))

    def load_start(j, k):
        di, dv = load_descs(j, k)
        di.start()
        dv.start()

    def load_wait(j, k):
        di, dv = load_descs(j, k)
        di.wait()
        dv.wait()

    def scat_start(k):
        pltpu.async_copy(vbuf.at[k], pool_sh.at[0].at[idxbuf.at[k, 0]],
                         semC[k], add=True)
        pltpu.async_copy(vbuf.at[k], pool_sh.at[1].at[idxbuf.at[k, 1]],
                         semC[k], add=True)

    def scat_wait(k):
        pltpu.make_async_copy(vbuf.at[k], pool_sh.at[0].at[idxbuf.at[k, 0]],
                              semC[k]).wait()
        pltpu.make_async_copy(vbuf.at[k], pool_sh.at[1].at[idxbuf.at[k, 1]],
                              semC[k]).wait()

    load_start(base + 0, 0)
    load_start(base + 1, 1)

    def step(u, carry):
        for k in range(3):
            t = 3 * u + k
            load_wait(base + t, k)
            scat_start(k)

            @pl.when(t >= 1)
            def _():
                scat_wait((k + 2) % 3)

            @pl.when(t + 2 < CPT)
            def _():
                load_start(base + t + 2, (k + 2) % 3)
        return carry
    lax.fori_loop(0, CPT // 3, step, 0)
    scat_wait((CPT - 1) % 3)

    @pl.when(s < CREM)
    def _():
        jx = NS * CPT + s
        load_start(jx, 0)
        load_wait(jx, 0)
        scat_start(0)
        scat_wait(0)

    plsc.subcore_barrier()

    for l in range(DH // 16):
        acc[pl.ds(l * 16, 16)] = jnp.zeros((16,), jnp.float32)
    for p in range(2):
        for q in range(RQ):
            r0 = s * ROWS_PT + q * RS
            pltpu.sync_copy(pool_sh.at[p, pl.ds(r0, RS)],
                            vbuf.at[0, pl.ds(0, RS)])
            if p == 0:
                def acc_row(r, carry):
                    for l in range(DH // 16):
                        sl = pl.ds(l * 16, 16)
                        acc[sl] = acc[sl] + vbuf[0, r, sl]
                    return carry
                lax.fori_loop(0, RS, acc_row, 0)
            pltpu.sync_copy(vbuf.at[0, pl.ds(0, RS)],
                            out.at[p, pl.ds(r0, RS), pl.ds(f0, DH)])
    pltpu.sync_copy(acc, totp.at[s, pl.ds(f0, DH)])


_pools_call = functools.partial(
    pl.kernel,
    out_type=[jax.ShapeDtypeStruct((NC, N_ROWS, D), jnp.float32),
              jax.ShapeDtypeStruct((NS, D), jnp.float32)],
    mesh=_mesh(),
    compiler_params=pltpu.CompilerParams(use_tc_tiling_on_sc=False),
    scratch_types=[
        pltpu.VMEM((3, 2, CH), jnp.int32),
        pltpu.VMEM((3, CH, DH), jnp.float32),
        pltpu.VMEM((DH,), jnp.float32),
        pltpu.VMEM_SHARED((2, N_ROWS, DH), jnp.float32),
        pltpu.SemaphoreType.DMA,
        pltpu.SemaphoreType.DMA,
        pltpu.SemaphoreType.DMA,
        pltpu.SemaphoreType.DMA,
        pltpu.SemaphoreType.DMA,
        pltpu.SemaphoreType.DMA,
    ],
)(_pools_body)


_SB = 10
_SR = N_ROWS // _SB


def _small_body(rp_ref, cp_ref, w1_ref, w2_ref, w3_ref, bias_ref, totp_ref,
                p1_ref, p2_ref):
    tot = jnp.sum(totp_ref[...], axis=0, keepdims=True)
    bsum = bias_ref[0] + bias_ref[1] + bias_ref[2] + bias_ref[3]
    tvec = jax.lax.dot_general(
        tot, w3_ref[...], (((1,), (0,)), ((), ())),
        precision=lax.Precision.HIGHEST,
        preferred_element_type=jnp.float32) + bsum
    p1_ref[...] = jax.lax.dot_general(
        rp_ref[...], w1_ref[...], (((1,), (0,)), ((), ())),
        precision=lax.Precision.HIGHEST,
        preferred_element_type=jnp.float32) + tvec
    p2_ref[...] = jax.lax.dot_general(
        cp_ref[...], w2_ref[...], (((1,), (0,)), ((), ())),
        precision=lax.Precision.HIGHEST, preferred_element_type=jnp.float32)


def _small_call(rp, cp, w1, w2, w3, bias, totp):
    return pl.pallas_call(
        _small_body,
        grid=(_SB,),
        in_specs=[
            pl.BlockSpec((_SR, D), lambda i: (i, 0)),
            pl.BlockSpec((_SR, D), lambda i: (i, 0)),
            pl.BlockSpec((D, D), lambda i: (0, 0)),
            pl.BlockSpec((D, D), lambda i: (0, 0)),
            pl.BlockSpec((D, D), lambda i: (0, 0)),
            pl.BlockSpec(memory_space=pltpu.SMEM),
            pl.BlockSpec((NS, D), lambda i: (0, 0)),
        ],
        out_specs=[
            pl.BlockSpec((_SR, D), lambda i: (i, 0)),
            pl.BlockSpec((_SR, D), lambda i: (i, 0)),
        ],
        out_shape=[
            jax.ShapeDtypeStruct((N_ROWS, D), jnp.float32),
            jax.ShapeDtypeStruct((N_ROWS, D), jnp.float32),
        ],
    )(rp, cp, w1, w2, w3, bias, totp)


def _gather_body(idx2d, p1, p2, m, y_out, idxbuf, buf1, buf2, mbuf,
                 semI0, semI1, semI2, semI3, semA0, semA1, semB0, semB1,
                 semM0, semM1, semS0, semS1):
    c = lax.axis_index("c")
    s = lax.axis_index("s")
    wid = s * NC + c
    base = wid * GPT
    semI = (semI0, semI1, semI2, semI3)
    semA = (semA0, semA1)
    semB = (semB0, semB1)
    semM = (semM0, semM1)
    semS = (semS0, semS1)

    def i_desc(j, ks):
        return pltpu.make_async_copy(idx2d.at[:, j], idxbuf.at[ks], semI[ks])

    def g_descs(j, ks, k):
        return (pltpu.make_async_copy(p1.at[idxbuf.at[ks, 0]], buf1.at[k],
                                      semA[k]),
                pltpu.make_async_copy(p2.at[idxbuf.at[ks, 1]], buf2.at[k],
                                      semB[k]))

    def g_start(j, ks, k):
        d1, d2 = g_descs(j, ks, k)
        d1.start()
        d2.start()

    def g_wait(j, ks, k):
        d1, d2 = g_descs(j, ks, k)
        d1.wait()
        d2.wait()

    def m_desc(j, k):
        return pltpu.make_async_copy(m.at[pl.ds(j * CH, CH)], mbuf.at[k],
                                     semM[k])

    def s_desc(j, k):
        return pltpu.make_async_copy(mbuf.at[k], y_out.at[pl.ds(j * CH, CH)],
                                     semS[k])

    def add_chunk(k):
        def add_row(r, carry):
            for l in range(D // 16):
                sl = pl.ds(l * 16, 16)
                plsc.addupdate(mbuf.at[k, r, sl],
                               buf1[k, r, sl] + buf2[k, r, sl])
            return carry
        lax.fori_loop(0, CH, add_row, 0)

    for ks in range(4):
        i_desc(base + ks, ks).start()
    i_desc(base + 0, 0).wait()
    g_start(base + 0, 0, 0)
    i_desc(base + 1, 1).wait()
    g_start(base + 1, 1, 1)
    m_desc(base + 0, 0).start()

    def step(u, carry):
        for j in range(4):
            k = j % 2
            t = 4 * u + j
            g_wait(base + t, j, k)
            m_desc(base + t, k).wait()
            add_chunk(k)
            i_desc(base + t + 2, (j + 2) % 4).wait()
            g_start(base + t + 2, (j + 2) % 4, k)
            s_desc(base + t, k).start()

            @pl.when(t >= 1)
            def _():
                s_desc(base + t - 1, 1 - k).wait()
            m_desc(base + t + 1, 1 - k).start()

            @pl.when(t + 4 < GPT)
            def _():
                i_desc(base + t + 4, j).start()
        return carry
    lax.fori_loop(0, GPT // 4, step, 0)

    for t, j in ((GPT - 2, (GPT - 2) % 4), (GPT - 1, (GPT - 1) % 4)):
        k = j % 2
        g_wait(base + t, j, k)
        m_desc(base + t, k).wait()
        add_chunk(k)
        s_desc(base + t, k).start()
        s_desc(base + t - 1, 1 - k).wait()
        if t + 1 < GPT:
            m_desc(base + t + 1, 1 - k).start()
    s_desc(base + GPT - 1, (GPT - 1) % 2).wait()

    @pl.when(wid < GREM)
    def _():
        jx = NW * GPT + wid
        i_desc(jx, 0).start()
        i_desc(jx, 0).wait()
        g_start(jx, 0, 0)
        m_desc(jx, 0).start()
        g_wait(jx, 0, 0)
        m_desc(jx, 0).wait()
        add_chunk(0)
        s_desc(jx, 0).start()
        s_desc(jx, 0).wait()


_gather_call = functools.partial(
    pl.kernel,
    out_type=jax.ShapeDtypeStruct((E, D), jnp.float32),
    mesh=_mesh(),
    compiler_params=pltpu.CompilerParams(use_tc_tiling_on_sc=False),
    scratch_types=[
        pltpu.VMEM((4, 2, CH), jnp.int32),
        pltpu.VMEM((2, CH, D), jnp.float32),
        pltpu.VMEM((2, CH, D), jnp.float32),
        pltpu.VMEM((2, CH, D), jnp.float32),
        pltpu.SemaphoreType.DMA,
        pltpu.SemaphoreType.DMA,
        pltpu.SemaphoreType.DMA,
        pltpu.SemaphoreType.DMA,
        pltpu.SemaphoreType.DMA,
        pltpu.SemaphoreType.DMA,
        pltpu.SemaphoreType.DMA,
        pltpu.SemaphoreType.DMA,
        pltpu.SemaphoreType.DMA,
        pltpu.SemaphoreType.DMA,
        pltpu.SemaphoreType.DMA,
        pltpu.SemaphoreType.DMA,
    ],
)(_gather_body)


_MB = 2560
_MG = E // _MB


def _mat_body(v_ref, w0_ref, y_ref):
    y_ref[...] = jax.lax.dot_general(
        v_ref[...], w0_ref[...], (((1,), (0,)), ((), ())),
        preferred_element_type=jnp.float32)


def _mat_call(values, w0):
    return pl.pallas_call(
        _mat_body,
        grid=(_MG,),
        in_specs=[
            pl.BlockSpec((_MB, D), lambda i: (i, 0)),
            pl.BlockSpec((D, D), lambda i: (0, 0)),
        ],
        out_specs=pl.BlockSpec((_MB, D), lambda i: (i, 0)),
        out_shape=jax.ShapeDtypeStruct((E, D), jnp.float32),
    )(values, w0)


def kernel(values, edge_index, W, bias):
    idx2d = edge_index.reshape(2, NB, CH)
    m = _mat_call(values, W[0])
    pools, totp = _pools_call(values, idx2d)
    p1, p2 = _small_call(pools[0], pools[1], W[1], W[2], W[3], bias, totp)
    return _gather_call(idx2d, p1, p2, m)
